# Initial kernel scaffold; baseline (speedup 1.0000x reference)
import jax
import jax.numpy as jnp
from jax.experimental import pallas as pl


def kernel(inputs, W):
    raise NotImplementedError("write your pallas kernel here")


# speedup vs baseline: 1.2877x; 1.2877x over previous
import functools

import jax
import jax.numpy as jnp
from jax import lax
from jax.experimental import pallas as pl
from jax.experimental.pallas import tpu as pltpu
from jax.experimental.pallas import tpu_sc as plsc

_N = 8192
_D = 32
_B = 8192
_RB = 256
_G = _B // _RB
_CC = 0.25


_HALF = _N // 2


def _assign_body(x_ref, w_ref, x2_ref, w2_ref, idx_ref, enc_ref, cnt_ref,
                 perp_ref):
    i = pl.program_id(0)
    x = x_ref[...]
    m = lax.dot_general(x, w_ref[...], (((1,), (1,)), ((), ())),
                        preferred_element_type=jnp.float32)
    d = (x2_ref[...] + w2_ref[...]) - 2.0 * m
    d0 = d[:, :_HALF]
    d1 = d[:, _HALF:]
    col = lax.broadcasted_iota(jnp.int32, d0.shape, 1)
    big = jnp.int32(2 ** 30)
    min0 = jnp.min(d0, axis=1, keepdims=True)
    idx0 = jnp.min(jnp.where(d0 == min0, col, big), axis=1)
    min1 = jnp.min(d1, axis=1, keepdims=True)
    idx1 = jnp.min(jnp.where(d1 == min1, col, big), axis=1) + _HALF
    carry = min0.astype(jnp.bfloat16).astype(jnp.float32)
    idx = jnp.where((min1 < carry)[:, 0], idx1, idx0)
    idx_ref[0, 0, :] = idx
    col = lax.broadcasted_iota(jnp.int32, d.shape, 1)
    onehot = (col == idx[:, None]).astype(jnp.float32)
    enc_ref[...] = onehot
    cnt = jnp.sum(onehot, axis=0, keepdims=True)

    @pl.when(i == 0)
    def _():
        cnt_ref[...] = cnt
        perp_ref[...] = jnp.zeros((1, 1), jnp.float32)

    @pl.when(i > 0)
    def _():
        cnt_ref[...] = cnt_ref[...] + cnt

    @pl.when(i == _G - 1)
    def _():
        p = cnt_ref[...] * (1.0 / _B)
        perp_ref[...] = jnp.exp(-jnp.sum(p * jnp.log(p + 1e-10),
                                         keepdims=True))


_assign = pl.pallas_call(
    _assign_body,
    grid=(_G,),
    in_specs=[
        pl.BlockSpec((_RB, _D), lambda i: (i, 0)),
        pl.BlockSpec((_N, _D), lambda i: (0, 0)),
        pl.BlockSpec((_RB, 1), lambda i: (i, 0)),
        pl.BlockSpec((1, _N), lambda i: (0, 0)),
    ],
    out_specs=[
        pl.BlockSpec((1, 1, _RB), lambda i: (i, 0, 0)),
        pl.BlockSpec((_RB, _N), lambda i: (i, 0)),
        pl.BlockSpec((1, _N), lambda i: (0, 0)),
        pl.BlockSpec((1, 1), lambda i: (0, 0)),
    ],
    out_shape=[
        jax.ShapeDtypeStruct((_G, 1, _RB), jnp.int32),
        jax.ShapeDtypeStruct((_B, _N), jnp.float32),
        jax.ShapeDtypeStruct((1, _N), jnp.float32),
        jax.ShapeDtypeStruct((1, 1), jnp.float32),
    ],
)


def _finalize_body(x_ref, q_ref, qst_ref, loss_ref):
    x = x_ref[...]
    q = q_ref[:, :_D]
    diff = q - x
    qst_ref[...] = x + diff
    loss_ref[...] = (1.0 + _CC) * (jnp.sum(diff * diff, keepdims=True)
                                   / (_B * _D))


_finalize = pl.pallas_call(
    _finalize_body,
    out_shape=[
        jax.ShapeDtypeStruct((_B, _D), jnp.float32),
        jax.ShapeDtypeStruct((1, 1), jnp.float32),
    ],
)


_DP = 128


def _sc_gather(table, idx):
    info = plsc.get_sparse_core_info()
    nw = info.num_cores * info.num_subcores
    bpw = _B // nw
    mesh = plsc.VectorSubcoreMesh(core_axis_name="c", subcore_axis_name="s")

    @functools.partial(
        pl.kernel, mesh=mesh,
        out_type=jax.ShapeDtypeStruct((_B, _DP), jnp.float32),
        scratch_types=[
            pltpu.VMEM((bpw,), jnp.int32),
            pltpu.VMEM((bpw, _DP), jnp.float32),
            pltpu.SemaphoreType.DMA,
        ],
    )
    def k(table_hbm, idx_hbm, out_hbm, idx_v, rows_v, sem):
        wid = lax.axis_index("s") * info.num_cores + lax.axis_index("c")
        base = wid * bpw
        pltpu.sync_copy(idx_hbm.at[pl.ds(base, bpw)], idx_v)
        pltpu.async_copy(table_hbm.at[idx_v], rows_v, sem).wait()
        pltpu.sync_copy(rows_v, out_hbm.at[pl.ds(base, bpw)])

    return k(table, idx)


def kernel(inputs, W):
    x = inputs.reshape(_B, _D)
    x2 = jnp.sum(inputs ** 2, axis=2).reshape(_B, 1)
    w2 = jnp.sum(W ** 2, axis=1)
    xb = x.astype(jnp.bfloat16)
    wb = W.astype(jnp.bfloat16)
    idx3, enc, _cnt, perp = _assign(xb, wb, x2, w2.reshape(1, _N))
    idx = idx3.reshape(_B)
    w_pad = jnp.pad(W, ((0, 0), (0, _DP - _D)))
    q_pad = _sc_gather(w_pad, idx)
    qst, loss = _finalize(x, q_pad)
    return (loss.reshape(()), qst.reshape(inputs.shape), perp.reshape(()),
            enc, idx)


# speedup vs baseline: 1.4333x; 1.1131x over previous
import functools

import jax
import jax.numpy as jnp
from jax import lax
from jax.experimental import pallas as pl
from jax.experimental.pallas import tpu as pltpu
from jax.experimental.pallas import tpu_sc as plsc

_N = 8192
_D = 32
_B = 8192
_RB = 256
_G = _B // _RB
_CC = 0.25


_HALF = _N // 2


def _assign_body(x_ref, w_ref, x2_ref, w2_ref, idx_ref, enc_ref, cnt_ref,
                 perp_ref):
    i = pl.program_id(0)
    x = x_ref[...]
    m = lax.dot_general(x, w_ref[...], (((1,), (1,)), ((), ())),
                        preferred_element_type=jnp.float32)
    d = (x2_ref[...] + w2_ref[...]) - 2.0 * m
    d0 = d[:, :_HALF]
    d1 = d[:, _HALF:]
    col = lax.broadcasted_iota(jnp.int32, d0.shape, 1)
    big = jnp.int32(2 ** 30)
    min0 = jnp.min(d0, axis=1, keepdims=True)
    idx0 = jnp.min(jnp.where(d0 == min0, col, big), axis=1)
    min1 = jnp.min(d1, axis=1, keepdims=True)
    idx1 = jnp.min(jnp.where(d1 == min1, col, big), axis=1) + _HALF
    carry = min0.astype(jnp.bfloat16).astype(jnp.float32)
    idx = jnp.where((min1 < carry)[:, 0], idx1, idx0)
    idx_ref[0, 0, :] = idx
    col = lax.broadcasted_iota(jnp.int32, d.shape, 1)
    onehot = (col == idx[:, None]).astype(jnp.float32)
    enc_ref[...] = onehot
    ones = jnp.ones((1, _RB), jnp.float32)
    cnt = lax.dot_general(ones, onehot, (((1,), (0,)), ((), ())),
                          preferred_element_type=jnp.float32)

    @pl.when(i == 0)
    def _():
        cnt_ref[...] = cnt
        perp_ref[...] = jnp.zeros((1, 1), jnp.float32)

    @pl.when(i > 0)
    def _():
        cnt_ref[...] = cnt_ref[...] + cnt

    @pl.when(i == _G - 1)
    def _():
        p = cnt_ref[...] * (1.0 / _B)
        perp_ref[...] = jnp.exp(-jnp.sum(p * jnp.log(p + 1e-10),
                                         keepdims=True))


_assign = pl.pallas_call(
    _assign_body,
    grid=(_G,),
    in_specs=[
        pl.BlockSpec((_RB, _D), lambda i: (i, 0)),
        pl.BlockSpec((_N, _D), lambda i: (0, 0)),
        pl.BlockSpec((_RB, 1), lambda i: (i, 0)),
        pl.BlockSpec((1, _N), lambda i: (0, 0)),
    ],
    out_specs=[
        pl.BlockSpec((1, 1, _RB), lambda i: (i, 0, 0)),
        pl.BlockSpec((_RB, _N), lambda i: (i, 0)),
        pl.BlockSpec((1, _N), lambda i: (0, 0)),
        pl.BlockSpec((1, 1), lambda i: (0, 0)),
    ],
    out_shape=[
        jax.ShapeDtypeStruct((_G, 1, _RB), jnp.int32),
        jax.ShapeDtypeStruct((_B, _N), jnp.float32),
        jax.ShapeDtypeStruct((1, _N), jnp.float32),
        jax.ShapeDtypeStruct((1, 1), jnp.float32),
    ],
)


def _finalize_body(x_ref, q_ref, qst_ref, loss_ref):
    x = x_ref[...]
    q = q_ref[:, :_D]
    diff = q - x
    qst_ref[...] = x + diff
    loss_ref[...] = (1.0 + _CC) * (jnp.sum(diff * diff, keepdims=True)
                                   / (_B * _D))


_finalize = pl.pallas_call(
    _finalize_body,
    out_shape=[
        jax.ShapeDtypeStruct((_B, _D), jnp.float32),
        jax.ShapeDtypeStruct((1, 1), jnp.float32),
    ],
)


_DP = 128


def _sc_gather(table, idx):
    info = plsc.get_sparse_core_info()
    nw = info.num_cores * info.num_subcores
    bpw = _B // nw
    mesh = plsc.VectorSubcoreMesh(core_axis_name="c", subcore_axis_name="s")

    @functools.partial(
        pl.kernel, mesh=mesh,
        out_type=jax.ShapeDtypeStruct((_B, _DP), jnp.float32),
        scratch_types=[
            pltpu.VMEM((bpw,), jnp.int32),
            pltpu.VMEM((bpw, _DP), jnp.float32),
            pltpu.SemaphoreType.DMA,
        ],
    )
    def k(table_hbm, idx_hbm, out_hbm, idx_v, rows_v, sem):
        wid = lax.axis_index("s") * info.num_cores + lax.axis_index("c")
        base = wid * bpw
        pltpu.sync_copy(idx_hbm.at[pl.ds(base, bpw)], idx_v)
        pltpu.async_copy(table_hbm.at[idx_v], rows_v, sem).wait()
        pltpu.sync_copy(rows_v, out_hbm.at[pl.ds(base, bpw)])

    return k(table, idx)


def kernel(inputs, W):
    x = inputs.reshape(_B, _D)
    x2 = jnp.sum(inputs ** 2, axis=2).reshape(_B, 1)
    w2 = jnp.sum(W ** 2, axis=1)
    xb = x.astype(jnp.bfloat16)
    wb = W.astype(jnp.bfloat16)
    idx3, enc, _cnt, perp = _assign(xb, wb, x2, w2.reshape(1, _N))
    idx = idx3.reshape(_B)
    w_pad = jnp.pad(W, ((0, 0), (0, _DP - _D)))
    q_pad = _sc_gather(w_pad, idx)
    qst, loss = _finalize(x, q_pad)
    return (loss.reshape(()), qst.reshape(inputs.shape), perp.reshape(()),
            enc, idx)


# speedup vs baseline: 1.4690x; 1.0249x over previous
import functools

import jax
import jax.numpy as jnp
from jax import lax
from jax.experimental import pallas as pl
from jax.experimental.pallas import tpu as pltpu
from jax.experimental.pallas import tpu_sc as plsc

_N = 8192
_D = 32
_B = 8192
_RB = 512
_G = _B // _RB
_CC = 0.25


_HALF = _N // 2


def _assign_body(x_ref, w_ref, x2_ref, w2_ref, idx_ref, enc_ref, cnt_ref,
                 perp_ref):
    i = pl.program_id(0)
    x = x_ref[...]
    m = lax.dot_general(x, w_ref[...], (((1,), (1,)), ((), ())),
                        preferred_element_type=jnp.float32)
    d = (x2_ref[...] + w2_ref[...]) - 2.0 * m
    d0 = d[:, :_HALF]
    d1 = d[:, _HALF:]
    col = lax.broadcasted_iota(jnp.int32, d0.shape, 1)
    big = jnp.int32(2 ** 30)
    min0 = jnp.min(d0, axis=1, keepdims=True)
    idx0 = jnp.min(jnp.where(d0 == min0, col, big), axis=1)
    min1 = jnp.min(d1, axis=1, keepdims=True)
    idx1 = jnp.min(jnp.where(d1 == min1, col, big), axis=1) + _HALF
    carry = min0.astype(jnp.bfloat16).astype(jnp.float32)
    idx = jnp.where((min1 < carry)[:, 0], idx1, idx0)
    idx_ref[0, 0, :] = idx
    col = lax.broadcasted_iota(jnp.int32, d.shape, 1)
    onehot = (col == idx[:, None]).astype(jnp.float32)
    enc_ref[...] = onehot
    ones = jnp.ones((1, _RB), jnp.float32)
    cnt = lax.dot_general(ones, onehot, (((1,), (0,)), ((), ())),
                          preferred_element_type=jnp.float32)

    @pl.when(i == 0)
    def _():
        cnt_ref[...] = cnt
        perp_ref[...] = jnp.zeros((1, 1), jnp.float32)

    @pl.when(i > 0)
    def _():
        cnt_ref[...] = cnt_ref[...] + cnt

    @pl.when(i == _G - 1)
    def _():
        p = cnt_ref[...] * (1.0 / _B)
        perp_ref[...] = jnp.exp(-jnp.sum(p * jnp.log(p + 1e-10),
                                         keepdims=True))


_assign = pl.pallas_call(
    _assign_body,
    grid=(_G,),
    in_specs=[
        pl.BlockSpec((_RB, _D), lambda i: (i, 0)),
        pl.BlockSpec((_N, _D), lambda i: (0, 0)),
        pl.BlockSpec((_RB, 1), lambda i: (i, 0)),
        pl.BlockSpec((1, _N), lambda i: (0, 0)),
    ],
    out_specs=[
        pl.BlockSpec((1, 1, _RB), lambda i: (i, 0, 0)),
        pl.BlockSpec((_RB, _N), lambda i: (i, 0)),
        pl.BlockSpec((1, _N), lambda i: (0, 0)),
        pl.BlockSpec((1, 1), lambda i: (0, 0)),
    ],
    out_shape=[
        jax.ShapeDtypeStruct((_G, 1, _RB), jnp.int32),
        jax.ShapeDtypeStruct((_B, _N), jnp.float32),
        jax.ShapeDtypeStruct((1, _N), jnp.float32),
        jax.ShapeDtypeStruct((1, 1), jnp.float32),
    ],
)


def _finalize_body(x_ref, q_ref, qst_ref, loss_ref):
    x = x_ref[...]
    q = q_ref[:, :_D]
    diff = q - x
    qst_ref[...] = x + diff
    loss_ref[...] = (1.0 + _CC) * (jnp.sum(diff * diff, keepdims=True)
                                   / (_B * _D))


_finalize = pl.pallas_call(
    _finalize_body,
    out_shape=[
        jax.ShapeDtypeStruct((_B, _D), jnp.float32),
        jax.ShapeDtypeStruct((1, 1), jnp.float32),
    ],
)


_DP = 128


def _sc_gather(table, idx):
    info = plsc.get_sparse_core_info()
    nw = info.num_cores * info.num_subcores
    bpw = _B // nw
    mesh = plsc.VectorSubcoreMesh(core_axis_name="c", subcore_axis_name="s")

    @functools.partial(
        pl.kernel, mesh=mesh,
        out_type=jax.ShapeDtypeStruct((_B, _DP), jnp.float32),
        scratch_types=[
            pltpu.VMEM((bpw,), jnp.int32),
            pltpu.VMEM((bpw, _DP), jnp.float32),
            pltpu.SemaphoreType.DMA,
        ],
    )
    def k(table_hbm, idx_hbm, out_hbm, idx_v, rows_v, sem):
        wid = lax.axis_index("s") * info.num_cores + lax.axis_index("c")
        base = wid * bpw
        pltpu.sync_copy(idx_hbm.at[pl.ds(base, bpw)], idx_v)
        pltpu.async_copy(table_hbm.at[idx_v], rows_v, sem).wait()
        pltpu.sync_copy(rows_v, out_hbm.at[pl.ds(base, bpw)])

    return k(table, idx)


def kernel(inputs, W):
    x = inputs.reshape(_B, _D)
    x2 = jnp.sum(inputs ** 2, axis=2).reshape(_B, 1)
    w2 = jnp.sum(W ** 2, axis=1)
    xb = x.astype(jnp.bfloat16)
    wb = W.astype(jnp.bfloat16)
    idx3, enc, _cnt, perp = _assign(xb, wb, x2, w2.reshape(1, _N))
    idx = idx3.reshape(_B)
    w_pad = jnp.pad(W, ((0, 0), (0, _DP - _D)))
    q_pad = _sc_gather(w_pad, idx)
    qst, loss = _finalize(x, q_pad)
    return (loss.reshape(()), qst.reshape(inputs.shape), perp.reshape(()),
            enc, idx)


# speedup vs baseline: 1.5165x; 1.0323x over previous
import functools

import jax
import jax.numpy as jnp
from jax import lax
from jax.experimental import pallas as pl
from jax.experimental.pallas import tpu as pltpu
from jax.experimental.pallas import tpu_sc as plsc

_N = 8192
_D = 32
_B = 8192
_RB = 512
_G = _B // _RB
_CC = 0.25


_HALF = _N // 2


def _assign_body(x_ref, w_ref, x2_ref, w2_ref, idx_ref, enc_ref, cnt_ref,
                 perp_ref):
    i = pl.program_id(0)
    x = x_ref[...].astype(jnp.bfloat16)
    w = w_ref[...].astype(jnp.bfloat16)
    m = lax.dot_general(x, w, (((1,), (1,)), ((), ())),
                        preferred_element_type=jnp.float32)
    d = (x2_ref[...] + w2_ref[...]) - 2.0 * m
    d0 = d[:, :_HALF]
    d1 = d[:, _HALF:]
    col = lax.broadcasted_iota(jnp.int32, d0.shape, 1)
    big = jnp.int32(2 ** 30)
    min0 = jnp.min(d0, axis=1, keepdims=True)
    idx0 = jnp.min(jnp.where(d0 == min0, col, big), axis=1)
    min1 = jnp.min(d1, axis=1, keepdims=True)
    idx1 = jnp.min(jnp.where(d1 == min1, col, big), axis=1) + _HALF
    carry = min0.astype(jnp.bfloat16).astype(jnp.float32)
    idx = jnp.where((min1 < carry)[:, 0], idx1, idx0)
    idx_ref[0, 0, :] = idx
    col = lax.broadcasted_iota(jnp.int32, d.shape, 1)
    onehot = (col == idx[:, None]).astype(jnp.float32)
    enc_ref[...] = onehot
    ones = jnp.ones((1, _RB), jnp.float32)
    cnt = lax.dot_general(ones, onehot, (((1,), (0,)), ((), ())),
                          preferred_element_type=jnp.float32)

    @pl.when(i == 0)
    def _():
        cnt_ref[...] = cnt
        perp_ref[...] = jnp.zeros((1, 1), jnp.float32)

    @pl.when(i > 0)
    def _():
        cnt_ref[...] = cnt_ref[...] + cnt

    @pl.when(i == _G - 1)
    def _():
        p = cnt_ref[...] * (1.0 / _B)
        perp_ref[...] = jnp.exp(-jnp.sum(p * jnp.log(p + 1e-10),
                                         keepdims=True))


_assign = pl.pallas_call(
    _assign_body,
    grid=(_G,),
    in_specs=[
        pl.BlockSpec((_RB, _D), lambda i: (i, 0)),
        pl.BlockSpec((_N, _D), lambda i: (0, 0)),
        pl.BlockSpec((_RB, 1), lambda i: (i, 0)),
        pl.BlockSpec((1, _N), lambda i: (0, 0)),
    ],
    out_specs=[
        pl.BlockSpec((1, 1, _RB), lambda i: (i, 0, 0)),
        pl.BlockSpec((_RB, _N), lambda i: (i, 0)),
        pl.BlockSpec((1, _N), lambda i: (0, 0)),
        pl.BlockSpec((1, 1), lambda i: (0, 0)),
    ],
    out_shape=[
        jax.ShapeDtypeStruct((_G, 1, _RB), jnp.int32),
        jax.ShapeDtypeStruct((_B, _N), jnp.float32),
        jax.ShapeDtypeStruct((1, _N), jnp.float32),
        jax.ShapeDtypeStruct((1, 1), jnp.float32),
    ],
)


def _finalize_body(x_ref, q_ref, qst_ref, loss_ref):
    x = x_ref[...]
    q = q_ref[:, :_D]
    diff = q - x
    qst_ref[...] = x + diff
    loss_ref[...] = (1.0 + _CC) * (jnp.sum(diff * diff, keepdims=True)
                                   / (_B * _D))


_finalize = pl.pallas_call(
    _finalize_body,
    out_shape=[
        jax.ShapeDtypeStruct((_B, _D), jnp.float32),
        jax.ShapeDtypeStruct((1, 1), jnp.float32),
    ],
)


_DP = 128


def _sc_gather(table, idx):
    info = plsc.get_sparse_core_info()
    nw = info.num_cores * info.num_subcores
    bpw = _B // nw
    mesh = plsc.VectorSubcoreMesh(core_axis_name="c", subcore_axis_name="s")

    @functools.partial(
        pl.kernel, mesh=mesh,
        out_type=jax.ShapeDtypeStruct((_B, _DP), jnp.float32),
        scratch_types=[
            pltpu.VMEM((bpw,), jnp.int32),
            pltpu.VMEM((bpw, _DP), jnp.float32),
            pltpu.SemaphoreType.DMA,
        ],
    )
    def k(table_hbm, idx_hbm, out_hbm, idx_v, rows_v, sem):
        wid = lax.axis_index("s") * info.num_cores + lax.axis_index("c")
        base = wid * bpw
        pltpu.sync_copy(idx_hbm.at[pl.ds(base, bpw)], idx_v)
        pltpu.async_copy(table_hbm.at[idx_v], rows_v, sem).wait()
        pltpu.sync_copy(rows_v, out_hbm.at[pl.ds(base, bpw)])

    return k(table, idx)


def kernel(inputs, W):
    x = inputs.reshape(_B, _D)
    x2 = jnp.sum(inputs ** 2, axis=2).reshape(_B, 1)
    w2 = jnp.sum(W ** 2, axis=1)
    idx3, enc, _cnt, perp = _assign(x, W, x2, w2.reshape(1, _N))
    idx = idx3.reshape(_B)
    w_pad = jnp.pad(W, ((0, 0), (0, _DP - _D)))
    q_pad = _sc_gather(w_pad, idx)
    qst, loss = _finalize(x, q_pad)
    return (loss.reshape(()), qst.reshape(inputs.shape), perp.reshape(()),
            enc, idx)
